# baseline (speedup 1.0000x reference)
import functools

import jax
import jax.numpy as jnp
from jax import lax
from jax.experimental import pallas as pl
from jax.experimental.pallas import tpu as pltpu
from jax.experimental.pallas import tpu_sc as plsc

_NC = 2
_NS = 16
_L = 16
_CH = 8
_NB = 3


def _cascade_sc(x_flat, table_flat, B, SL):
    NW = _NC * _NS
    flat = (B // NW) * SL
    chunk = flat // _CH
    gpc = chunk // (_L * SL)

    mesh = plsc.VectorSubcoreMesh(core_axis_name="c", subcore_axis_name="s")

    @functools.partial(
        pl.kernel,
        mesh=mesh,
        out_type=jax.ShapeDtypeStruct((B * SL,), jnp.float32),
        scratch_types=[
            pltpu.VMEM((flat,), jnp.int32),
        ] + [pltpu.VMEM((chunk,), jnp.float32) for _ in range(5)] + [
            pltpu.VMEM((flat,), jnp.float32),
            pltpu.VMEM_SHARED((1000 * 1000,), jnp.float32),
            pltpu.VMEM((20736,), jnp.float32),
            pltpu.VMEM((20736,), jnp.float32),
        ] + [pltpu.SemaphoreType.DMA for _ in range(9)],
        compiler_params=pltpu.CompilerParams(
            needs_layout_passes=False,
            skip_device_barrier=True,
            disable_bounds_checks=True,
        ),
    )
    def run(x_hbm, table_hbm, out_hbm, idx_v, *rest):
        bufs = rest[:5]
        out_v = rest[5]
        table_sp = rest[6]
        bounce = (rest[7], rest[8])
        sems = rest[9:14]
        osem, isem, tsem, t2sem = rest[14:18]
        sid = lax.axis_index("s")
        wid = sid * _NC + lax.axis_index("c")
        base = wid * flat
        lane = lax.iota(jnp.int32, _L) * SL

        def scan_chunk(k, buf):
            obase = k * chunk

            def group(g, _):
                gbase = g * (_L * SL)
                ivs = [lane + (gbase + l) for l in range(SL)]
                vs = [plsc.load_gather(buf, [iv]) for iv in ivs]
                rs = [1.0 / (1.0 + jnp.exp(-v)) for v in vs]
                cum = jnp.full((_L,), 1.0, jnp.float32)
                for l in range(SL):
                    plsc.store_scatter(out_v, [obase + ivs[l]], cum * rs[l])
                    cum = cum * (1.0 - rs[l])
                return 0

            lax.fori_loop(0, gpc, group, 0)

        def idx_copy(k):
            return pltpu.async_copy(
                x_hbm.at[pl.ds(base + k * chunk, chunk)],
                idx_v.at[pl.ds(k * chunk, chunk)], isem)

        def gather(k, src, b):
            return pltpu.async_copy(
                src.at[idx_v.at[pl.ds(k * chunk, chunk)]],
                bufs[b], sems[b])

        v_rows = 1000 * 1000
        bchunk = 20736
        nstg = 3
        tslice = bchunk * nstg
        tail = v_rows - _NS * tslice

        def _hbm_off(c):
            return sid * tslice + c * bchunk

        hb = [None] * nstg
        sb = [None] * nstg
        hb[0] = pltpu.async_copy(
            table_hbm.at[pl.ds(_hbm_off(0), bchunk)], bounce[0], tsem)
        ih = [idx_copy(k) for k in range(_CH)]
        for c in range(nstg):
            hb[c].wait()
            sb[c] = pltpu.async_copy(
                bounce[c % 2], table_sp.at[pl.ds(_hbm_off(c), bchunk)], t2sem)
            if c + 1 < nstg:
                if c >= 1:
                    sb[c - 1].wait()
                hb[c + 1] = pltpu.async_copy(
                    table_hbm.at[pl.ds(_hbm_off(c + 1), bchunk)],
                    bounce[(c + 1) % 2], tsem)
        for k in range(_CH):
            ih[k].wait()
        gath = [None] * _CH
        gath[3] = gather(3, table_hbm, 3)
        gath[7] = gather(7, table_hbm, 4)
        sb[nstg - 2].wait()
        sb[nstg - 1].wait()

        @pl.when(sid == 0)
        def _copy_tail():
            pltpu.async_copy(
                table_hbm.at[pl.ds(_NS * tslice, tail)],
                bounce[0].at[pl.ds(0, tail)], tsem).wait()
            pltpu.async_copy(
                bounce[0].at[pl.ds(0, tail)],
                table_sp.at[pl.ds(_NS * tslice, tail)], t2sem).wait()

        plsc.subcore_barrier()
        order = [0, 1, 2, 4, 5, 6, 3, 7]
        nsp = 6
        for i in range(2):
            gath[order[i]] = gather(order[i], table_sp, i % 3)
        outs = []
        for i, k in enumerate(order):
            if i + 2 < nsp:
                gath[order[i + 2]] = gather(order[i + 2], table_sp,
                                            (i + 2) % 3)
            gath[k].wait()
            scan_chunk(k, bufs[(i % 3) if i < nsp else (i - nsp + 3)])
            outs.append(pltpu.async_copy(
                out_v.at[pl.ds(k * chunk, chunk)],
                out_hbm.at[pl.ds(base + k * chunk, chunk)], osem))
        for h in outs:
            h.wait()

    return run(x_flat, table_flat)


def kernel(x, table):
    B, SL = x.shape
    out = _cascade_sc(x.reshape(-1), table.reshape(-1), B, SL)
    return out.reshape(B, SL)


# speedup vs baseline: 1.0062x; 1.0062x over previous
import functools

import jax
import jax.numpy as jnp
from jax import lax
from jax.experimental import pallas as pl
from jax.experimental.pallas import tpu as pltpu
from jax.experimental.pallas import tpu_sc as plsc

_NC = 2
_NS = 16
_L = 16
_CH = 8
_NB = 3


def _cascade_sc(x_flat, table_flat, B, SL):
    NW = _NC * _NS
    flat = (B // NW) * SL
    chunk = flat // _CH
    gpc = chunk // (_L * SL)

    mesh = plsc.VectorSubcoreMesh(core_axis_name="c", subcore_axis_name="s")

    @functools.partial(
        pl.kernel,
        mesh=mesh,
        out_type=jax.ShapeDtypeStruct((B * SL,), jnp.float32),
        scratch_types=[
            pltpu.VMEM((flat,), jnp.int32),
        ] + [pltpu.VMEM((chunk,), jnp.float32) for _ in range(_NB)] + [
            pltpu.VMEM((flat,), jnp.float32),
            pltpu.VMEM_SHARED((1000 * 1000,), jnp.float32),
            pltpu.VMEM((20800,), jnp.float32),
            pltpu.VMEM((20800,), jnp.float32),
        ] + [pltpu.SemaphoreType.DMA for _ in range(_NB + 4)],
        compiler_params=pltpu.CompilerParams(
            needs_layout_passes=False,
            skip_device_barrier=True,
            disable_bounds_checks=True,
        ),
    )
    def run(x_hbm, table_hbm, out_hbm, idx_v, *rest):
        bufs = rest[:_NB]
        out_v = rest[_NB]
        table_sp = rest[_NB + 1]
        bounce = (rest[_NB + 2], rest[_NB + 3])
        sems = rest[_NB + 4:2 * _NB + 4]
        osem, isem, tsem, t2sem = rest[2 * _NB + 4:2 * _NB + 8]
        sid = lax.axis_index("s")
        wid = sid * _NC + lax.axis_index("c")
        base = wid * flat
        lane = lax.iota(jnp.int32, _L) * SL

        def scan_chunk(k, buf):
            obase = k * chunk

            def group(g, _):
                gbase = g * (_L * SL)
                ivs = [lane + (gbase + l) for l in range(SL)]
                vs = [plsc.load_gather(buf, [iv]) for iv in ivs]
                rs = [1.0 / (1.0 + jnp.exp(-v)) for v in vs]
                cum = jnp.full((_L,), 1.0, jnp.float32)
                for l in range(SL):
                    plsc.store_scatter(out_v, [obase + ivs[l]], cum * rs[l])
                    cum = cum * (1.0 - rs[l])
                return 0

            lax.fori_loop(0, gpc, group, 0)

        def idx_copy(k):
            return pltpu.async_copy(
                x_hbm.at[pl.ds(base + k * chunk, chunk)],
                idx_v.at[pl.ds(k * chunk, chunk)], isem)

        def gather(k):
            return pltpu.async_copy(
                table_sp.at[idx_v.at[pl.ds(k * chunk, chunk)]],
                bufs[k % _NB], sems[k % _NB])

        v_rows = 1000 * 1000
        bchunk = 20800
        nstg = 3
        tslice = bchunk * nstg
        tail = v_rows - _NS * tslice

        def _hbm_off(c):
            return sid * tslice + c * bchunk

        hb = [None] * nstg
        sb = [None] * nstg
        hb[0] = pltpu.async_copy(
            table_hbm.at[pl.ds(_hbm_off(0), bchunk)], bounce[0], tsem)
        ih = [idx_copy(k) for k in range(_CH)]
        for c in range(nstg):
            hb[c].wait()
            sb[c] = pltpu.async_copy(
                bounce[c % 2], table_sp.at[pl.ds(_hbm_off(c), bchunk)], t2sem)
            if c + 1 < nstg:
                if c >= 1:
                    sb[c - 1].wait()
                hb[c + 1] = pltpu.async_copy(
                    table_hbm.at[pl.ds(_hbm_off(c + 1), bchunk)],
                    bounce[(c + 1) % 2], tsem)
        sb[nstg - 2].wait()
        sb[nstg - 1].wait()

        @pl.when(sid == 0)
        def _copy_tail():
            pltpu.async_copy(
                table_hbm.at[pl.ds(_NS * tslice, tail)],
                bounce[0].at[pl.ds(0, tail)], tsem).wait()
            pltpu.async_copy(
                bounce[0].at[pl.ds(0, tail)],
                table_sp.at[pl.ds(_NS * tslice, tail)], t2sem).wait()

        plsc.subcore_barrier()
        gath = [None] * _CH
        for k in range(_NB - 1):
            ih[k].wait()
            gath[k] = gather(k)
        outs = []
        for k in range(_CH):
            nxt = k + _NB - 1
            if nxt < _CH:
                ih[nxt].wait()
                gath[nxt] = gather(nxt)
            gath[k].wait()
            scan_chunk(k, bufs[k % _NB])
            outs.append(pltpu.async_copy(
                out_v.at[pl.ds(k * chunk, chunk)],
                out_hbm.at[pl.ds(base + k * chunk, chunk)], osem))
        for h in outs:
            h.wait()

    return run(x_flat, table_flat)


def kernel(x, table):
    B, SL = x.shape
    out = _cascade_sc(x.reshape(-1), table.reshape(-1), B, SL)
    return out.reshape(B, SL)
